# Initial kernel scaffold; baseline (speedup 1.0000x reference)
import jax
import jax.numpy as jnp
from jax.experimental import pallas as pl


def kernel(x, W0, gamma0, beta0, W1, gamma1, beta1, W2, gamma2, beta2, W3, gamma3, beta3, Wf, bf):
    raise NotImplementedError("write your pallas kernel here")


# speedup vs baseline: 3.9251x; 3.9251x over previous
import functools

import jax
import jax.numpy as jnp
from jax import lax
from jax.experimental import pallas as pl
from jax.experimental.pallas import tpu as pltpu
from jax.experimental.pallas import tpu_sc as plsc

K = 40
B = 4
N = 1024
R = B * N
CP = 128
NB = 256
NBE = 128
NTILES = 32
RPT = R // NTILES

_i32 = jnp.int32
_f32 = jnp.float32
_bf16 = jnp.bfloat16
INT_MIN = jnp.iinfo(jnp.int32).min


def _prep_body(hb_ref, hf_ref, dk_ref):
    hb = hb_ref[0]
    hf = hf_ref[0]
    dn = (((1,), (1,)), ((), ()))
    g = lax.dot_general(hb.astype(_bf16), hf.astype(_bf16), dn,
                        preferred_element_type=_f32)
    inner = -2.0 * g
    xxb = jnp.sum(hb * hb, axis=1, keepdims=True)
    sq = hf * hf
    ones = jnp.ones((8, CP), _f32)
    xx8 = lax.dot_general(ones, sq, dn, preferred_element_type=_f32,
                          precision=lax.Precision.HIGHEST)
    d = (-xxb - inner) - xx8[0:1, :]
    u = lax.bitcast_convert_type(d, _i32)
    key = jnp.where(u >= 0, u, u ^ jnp.int32(0x7FFFFFFF))
    dk_ref[...] = key[None]


def _prep(h):
    grid = (B, N // NB)
    return pl.pallas_call(
        _prep_body,
        grid=grid,
        in_specs=[
            pl.BlockSpec((1, NB, CP), lambda b, i: (b, i, 0)),
            pl.BlockSpec((1, N, CP), lambda b, i: (b, 0, 0)),
        ],
        out_specs=pl.BlockSpec((1, NB, N), lambda b, i: (b, i, 0)),
        out_shape=jax.ShapeDtypeStruct((B, N, N), _i32),
    )(h, h)


def _sc_body(dk_hbm, h_hbm, nbr_hbm, dbuf, dbufT, idxbuf, gbuf, sem):
    wid = lax.axis_index("s") * 2 + lax.axis_index("c")
    iota = lax.iota(_i32, 16)
    j0 = iota
    j1 = iota + 16
    j2 = iota + 32
    j3 = iota + 48

    def row_body(rl, carry):
        r = wid * RPT + rl
        bofs = (r // N) * N
        pltpu.sync_copy(dk_hbm.at[r], dbuf)

        def init_j(j, v):
            ch = dbuf[pl.ds(j * 16, 16)]
            plsc.store_scatter(dbufT, [iota * 64 + j], ch)
            return jnp.maximum(v, ch)

        v = lax.fori_loop(0, 64, init_j, jnp.zeros((16,), _i32) + INT_MIN)

        def sel_t(t, v):
            rowmax = jnp.max(v)
            lane = jnp.min(jnp.where(v == rowmax, iota, 16))
            base = lane * 64
            c0 = dbufT[pl.ds(base, 16)]
            c1 = dbufT[pl.ds(base + 16, 16)]
            c2 = dbufT[pl.ds(base + 32, 16)]
            c3 = dbufT[pl.ds(base + 48, 16)]
            cand = jnp.minimum(
                jnp.minimum(jnp.where(c0 == rowmax, j0, 64),
                            jnp.where(c1 == rowmax, j1, 64)),
                jnp.minimum(jnp.where(c2 == rowmax, j2, 64),
                            jnp.where(c3 == rowmax, j3, 64)))
            jstar = jnp.min(cand)
            gidx = jstar * 16 + lane
            zero = jnp.zeros((16,), _i32)
            plsc.store_scatter(idxbuf, [zero + t], zero + (gidx + bofs),
                               mask=iota == 0)
            plsc.store_scatter(dbufT, [zero + (base + jstar)], zero + INT_MIN,
                               mask=iota == 0)
            m0 = jnp.where(j0 == jstar, INT_MIN, c0)
            m1 = jnp.where(j1 == jstar, INT_MIN, c1)
            m2 = jnp.where(j2 == jstar, INT_MIN, c2)
            m3 = jnp.where(j3 == jstar, INT_MIN, c3)
            newmax = jnp.max(jnp.maximum(jnp.maximum(m0, m1),
                                         jnp.maximum(m2, m3)))
            return jnp.where(iota == lane, newmax, v)

        lax.fori_loop(0, K, sel_t, v)

        pltpu.async_copy(h_hbm.at[idxbuf], gbuf, sem).wait()
        pltpu.sync_copy(gbuf, nbr_hbm.at[r])
        return carry

    lax.fori_loop(0, RPT, row_body, 0)


def _sc_select_gather(dk2d, h2d):
    fn = pl.kernel(
        _sc_body,
        out_type=jax.ShapeDtypeStruct((R, K, CP), _f32),
        compiler_params=pltpu.CompilerParams(needs_layout_passes=False),
        mesh=plsc.VectorSubcoreMesh(core_axis_name="c", subcore_axis_name="s"),
        scratch_types=[
            pltpu.VMEM((N,), _i32),
            pltpu.VMEM((N,), _i32),
            pltpu.VMEM((K,), _i32),
            pltpu.VMEM((K, CP), _f32),
            pltpu.SemaphoreType.DMA,
        ],
    )
    return fn(dk2d, h2d)


def _edge_body(nbr_ref, h_ref, w_ref, m_ref, st_ref):
    i = pl.program_id(0)
    nbr = nbr_ref[...]
    ctr = h_ref[...]
    ctr3 = jnp.broadcast_to(ctr[:, None, :], nbr.shape)
    a = (nbr - ctr3).astype(_bf16)
    bb = ctr3.astype(_bf16)
    feat = jnp.concatenate([a, bb], axis=-1).reshape(NBE * K, 2 * CP)
    dn = (((1,), (0,)), ((), ()))
    y = lax.dot_general(feat, w_ref[...], dn, preferred_element_type=_f32)
    co = y.shape[1]
    m_ref[...] = jnp.max(y.reshape(NBE, K, co), axis=1)

    @pl.when(i == 0)
    def _():
        st_ref[...] = jnp.zeros_like(st_ref)

    st_ref[0:1, :] += jnp.sum(y, axis=0, keepdims=True)
    st_ref[1:2, :] += jnp.sum(y * y, axis=0, keepdims=True)


def _edge(nbr, h2d, wcat):
    co = wcat.shape[1]
    return pl.pallas_call(
        _edge_body,
        grid=(R // NBE,),
        in_specs=[
            pl.BlockSpec((NBE, K, CP), lambda i: (i, 0, 0)),
            pl.BlockSpec((NBE, CP), lambda i: (i, 0)),
            pl.BlockSpec((2 * CP, co), lambda i: (0, 0)),
        ],
        out_specs=[
            pl.BlockSpec((NBE, co), lambda i: (i, 0)),
            pl.BlockSpec((8, co), lambda i: (0, 0)),
        ],
        out_shape=[
            jax.ShapeDtypeStruct((R, co), _f32),
            jax.ShapeDtypeStruct((8, co), _f32),
        ],
    )(nbr, h2d, wcat)


def _norm_body(m_ref, st_ref, g_ref, b_ref, o_ref):
    bnk = _f32(R * K)
    mean = st_ref[0:1, :] / bnk
    var = st_ref[1:2, :] / bnk - mean * mean
    y = g_ref[...] * (m_ref[...] - mean) / jnp.sqrt(var + 1e-5) + b_ref[...]
    o_ref[...] = jnp.where(y > 0, y, 0.2 * y)


def _norm(m2d, st, gamma, beta):
    co = m2d.shape[1]
    return pl.pallas_call(
        _norm_body,
        grid=(R // NB,),
        in_specs=[
            pl.BlockSpec((NB, co), lambda i: (i, 0)),
            pl.BlockSpec((8, co), lambda i: (0, 0)),
            pl.BlockSpec((1, co), lambda i: (0, 0)),
            pl.BlockSpec((1, co), lambda i: (0, 0)),
        ],
        out_specs=pl.BlockSpec((NB, co), lambda i: (i, 0)),
        out_shape=jax.ShapeDtypeStruct((R, co), _f32),
    )(m2d, st, gamma, beta)


def _final_body(h1_ref, h2_ref, h3_ref, h4_ref, w1_ref, w2_ref, w3_ref,
                w4_ref, bf_ref, o_ref):
    b = pl.program_id(0)
    i = pl.program_id(1)
    dn = (((1,), (0,)), ((), ()))
    y = lax.dot_general(h1_ref[...].astype(_bf16), w1_ref[...], dn,
                        preferred_element_type=_f32)
    y += lax.dot_general(h2_ref[...].astype(_bf16), w2_ref[...], dn,
                         preferred_element_type=_f32)
    y += lax.dot_general(h3_ref[...].astype(_bf16), w3_ref[...], dn,
                         preferred_element_type=_f32)
    y += lax.dot_general(h4_ref[...].astype(_bf16), w4_ref[...], dn,
                         preferred_element_type=_f32)
    y += bf_ref[...]
    part = jnp.max(y, axis=0, keepdims=True)

    @pl.when(i == 0)
    def _():
        o_ref[pl.ds(b, 1), :] = part

    @pl.when(i != 0)
    def _():
        o_ref[pl.ds(b, 1), :] = jnp.maximum(o_ref[pl.ds(b, 1), :], part)


def _final(hs, wfs, bf_row):
    nblk = N // NB
    in_specs = []
    args = []
    for h in hs:
        co = h.shape[1]
        in_specs.append(
            pl.BlockSpec((NB, co), lambda b, i: (b * nblk + i, 0)))
        args.append(h)
    for w in wfs:
        ci = w.shape[0]
        in_specs.append(pl.BlockSpec((ci, 1024), lambda b, i: (0, 0)))
        args.append(w)
    in_specs.append(pl.BlockSpec((1, 1024), lambda b, i: (0, 0)))
    args.append(bf_row)
    return pl.pallas_call(
        _final_body,
        grid=(B, nblk),
        in_specs=in_specs,
        out_specs=pl.BlockSpec((B, 1024), lambda b, i: (0, 0)),
        out_shape=jax.ShapeDtypeStruct((B, 1024), _f32),
    )(*args)


def kernel(x, W0, gamma0, beta0, W1, gamma1, beta1, W2, gamma2, beta2,
           W3, gamma3, beta3, Wf, bf):
    layers = [(W0, gamma0, beta0), (W1, gamma1, beta1), (W2, gamma2, beta2),
              (W3, gamma3, beta3)]
    h3d = jnp.pad(x, ((0, 0), (0, 0), (0, CP - 3)))
    ci_real = 3
    hs = []
    cos = []
    for W, gamma, beta in layers:
        co = W.shape[0]
        cp = max(co, CP)
        wa = jnp.pad(W[:, :ci_real].T, ((0, CP - ci_real), (0, cp - co)))
        wb = jnp.pad(W[:, ci_real:].T, ((0, CP - ci_real), (0, cp - co)))
        wcat = jnp.concatenate([wa, wb], axis=0).astype(_bf16)
        h2d = h3d.reshape(R, CP)
        dk = _prep(h3d)
        nbr = _sc_select_gather(dk.reshape(R, N), h2d)
        m2d, st = _edge(nbr, h2d, wcat)
        gp = jnp.pad(gamma, (0, cp - co)).reshape(1, cp)
        bp = jnp.pad(beta, (0, cp - co)).reshape(1, cp)
        hn = _norm(m2d, st, gp, bp)
        hs.append(hn)
        cos.append(co)
        if cp > CP:
            break
        h3d = hn.reshape(B, N, cp)
        ci_real = co
    ofs = 0
    wfs = []
    for hh, co in zip(hs, cos):
        cp = hh.shape[1]
        wfs.append(jnp.pad(Wf[:, ofs:ofs + co].T,
                           ((0, cp - co), (0, 0))).astype(_bf16))
        ofs += co
    return _final(hs, wfs, bf.reshape(1, 1024))


# speedup vs baseline: 4.9818x; 1.2692x over previous
import functools

import jax
import jax.numpy as jnp
from jax import lax
from jax.experimental import pallas as pl
from jax.experimental.pallas import tpu as pltpu
from jax.experimental.pallas import tpu_sc as plsc

K = 40
B = 4
N = 1024
R = B * N
CP = 128
NB = 256
NBE = 128
NTILES = 32
RPT = R // NTILES

_i32 = jnp.int32
_f32 = jnp.float32
_bf16 = jnp.bfloat16
INT_MIN = jnp.iinfo(jnp.int32).min


def _prep_body(hb_ref, hf_ref, dk_ref):
    hb = hb_ref[0]
    hf = hf_ref[0]
    dn = (((1,), (1,)), ((), ()))
    g = lax.dot_general(hb.astype(_bf16), hf.astype(_bf16), dn,
                        preferred_element_type=_f32)
    inner = -2.0 * g
    xxb = jnp.sum(hb * hb, axis=1, keepdims=True)
    sq = hf * hf
    ones = jnp.ones((8, CP), _f32)
    xx8 = lax.dot_general(ones, sq, dn, preferred_element_type=_f32,
                          precision=lax.Precision.HIGHEST)
    d = (-xxb - inner) - xx8[0:1, :]
    u = lax.bitcast_convert_type(d, _i32)
    key = jnp.where(u >= 0, u, u ^ jnp.int32(0x7FFFFFFF))
    dk_ref[...] = key[None]


def _prep(h):
    grid = (B, N // NB)
    return pl.pallas_call(
        _prep_body,
        grid=grid,
        in_specs=[
            pl.BlockSpec((1, NB, CP), lambda b, i: (b, i, 0)),
            pl.BlockSpec((1, N, CP), lambda b, i: (b, 0, 0)),
        ],
        out_specs=pl.BlockSpec((1, NB, N), lambda b, i: (b, i, 0)),
        out_shape=jax.ShapeDtypeStruct((B, N, N), _i32),
    )(h, h)


def _sc_body(dk_hbm, h_hbm, nbr_hbm, dbufA, dbufB, dbufT, idxA, idxB,
             gbufA, gbufB, semDA, semDB, semGA, semGB, semWA, semWB):
    wid = lax.axis_index("s") * 2 + lax.axis_index("c")
    iota = lax.iota(_i32, 16)
    j0 = iota
    j1 = iota + 16
    j2 = iota + 32
    j3 = iota + 48
    base_r = wid * RPT

    def select(dbuf, idxbuf, r):
        bofs = (r // N) * N

        def init_j(j, v):
            ch = dbuf[pl.ds(j * 16, 16)]
            plsc.store_scatter(dbufT, [iota * 64 + j], ch)
            return jnp.maximum(v, ch)

        v = lax.fori_loop(0, 64, init_j, jnp.zeros((16,), _i32) + INT_MIN,
                          unroll=4)

        def sel_t(t, v):
            rowmax = jnp.max(v)
            lane = jnp.min(jnp.where(v == rowmax, iota, 16))
            base = lane * 64
            c0 = dbufT[pl.ds(base, 16)]
            c1 = dbufT[pl.ds(base + 16, 16)]
            c2 = dbufT[pl.ds(base + 32, 16)]
            c3 = dbufT[pl.ds(base + 48, 16)]
            cand = jnp.minimum(
                jnp.minimum(jnp.where(c0 == rowmax, j0, 64),
                            jnp.where(c1 == rowmax, j1, 64)),
                jnp.minimum(jnp.where(c2 == rowmax, j2, 64),
                            jnp.where(c3 == rowmax, j3, 64)))
            jstar = jnp.min(cand)
            gidx = jstar * 16 + lane
            zero = jnp.zeros((16,), _i32)
            plsc.store_scatter(idxbuf, [zero + t], zero + (gidx + bofs),
                               mask=iota == 0)
            plsc.store_scatter(dbufT, [zero + (base + jstar)], zero + INT_MIN,
                               mask=iota == 0)
            m0 = jnp.where(j0 == jstar, INT_MIN, c0)
            m1 = jnp.where(j1 == jstar, INT_MIN, c1)
            m2 = jnp.where(j2 == jstar, INT_MIN, c2)
            m3 = jnp.where(j3 == jstar, INT_MIN, c3)
            newmax = jnp.max(jnp.maximum(jnp.maximum(m0, m1),
                                         jnp.maximum(m2, m3)))
            return jnp.where(iota == lane, newmax, v)

        lax.fori_loop(0, K, sel_t, v)

    pltpu.async_copy(dk_hbm.at[base_r], dbufA, semDA)
    pltpu.async_copy(dk_hbm.at[base_r + 1], dbufB, semDB)
    NP = RPT // 2

    def pair_body(i, carry):
        ra = base_r + 2 * i
        rb = ra + 1

        pltpu.make_async_copy(dk_hbm.at[ra], dbufA, semDA).wait()
        select(dbufA, idxA, ra)

        @pl.when(i > 0)
        def _():
            pltpu.make_async_copy(gbufA, nbr_hbm.at[ra - 2], semWA).wait()

        pltpu.async_copy(h_hbm.at[idxA], gbufA, semGA)

        @pl.when(i < NP - 1)
        def _():
            pltpu.async_copy(dk_hbm.at[ra + 2], dbufA, semDA)

        pltpu.make_async_copy(dk_hbm.at[rb], dbufB, semDB).wait()
        select(dbufB, idxB, rb)

        @pl.when(i > 0)
        def _():
            pltpu.make_async_copy(gbufB, nbr_hbm.at[rb - 2], semWB).wait()

        pltpu.async_copy(h_hbm.at[idxB], gbufB, semGB)

        @pl.when(i < NP - 1)
        def _():
            pltpu.async_copy(dk_hbm.at[rb + 2], dbufB, semDB)

        pltpu.make_async_copy(h_hbm.at[idxA], gbufA, semGA).wait()
        pltpu.async_copy(gbufA, nbr_hbm.at[ra], semWA)
        pltpu.make_async_copy(h_hbm.at[idxB], gbufB, semGB).wait()
        pltpu.async_copy(gbufB, nbr_hbm.at[rb], semWB)
        return carry

    lax.fori_loop(0, NP, pair_body, 0)
    pltpu.make_async_copy(gbufA, nbr_hbm.at[base_r], semWA).wait()
    pltpu.make_async_copy(gbufB, nbr_hbm.at[base_r], semWB).wait()


def _sc_select_gather(dk2d, h2d):
    fn = pl.kernel(
        _sc_body,
        out_type=jax.ShapeDtypeStruct((R, K, CP), _f32),
        compiler_params=pltpu.CompilerParams(needs_layout_passes=False),
        mesh=plsc.VectorSubcoreMesh(core_axis_name="c", subcore_axis_name="s"),
        scratch_types=[
            pltpu.VMEM((N,), _i32),
            pltpu.VMEM((N,), _i32),
            pltpu.VMEM((N,), _i32),
            pltpu.VMEM((K,), _i32),
            pltpu.VMEM((K,), _i32),
            pltpu.VMEM((K, CP), _f32),
            pltpu.VMEM((K, CP), _f32),
            pltpu.SemaphoreType.DMA,
            pltpu.SemaphoreType.DMA,
            pltpu.SemaphoreType.DMA,
            pltpu.SemaphoreType.DMA,
            pltpu.SemaphoreType.DMA,
            pltpu.SemaphoreType.DMA,
        ],
    )
    return fn(dk2d, h2d)


def _edge_body(nbr_ref, h_ref, w_ref, m_ref, st_ref):
    i = pl.program_id(0)
    nbr = nbr_ref[...]
    ctr = h_ref[...]
    ctr3 = jnp.broadcast_to(ctr[:, None, :], nbr.shape)
    a = (nbr - ctr3).astype(_bf16)
    bb = ctr3.astype(_bf16)
    feat = jnp.concatenate([a, bb], axis=-1).reshape(NBE * K, 2 * CP)
    dn = (((1,), (0,)), ((), ()))
    y = lax.dot_general(feat, w_ref[...], dn, preferred_element_type=_f32)
    co = y.shape[1]
    m_ref[...] = jnp.max(y.reshape(NBE, K, co), axis=1)

    @pl.when(i == 0)
    def _():
        st_ref[...] = jnp.zeros_like(st_ref)

    st_ref[0:1, :] += jnp.sum(y, axis=0, keepdims=True)
    st_ref[1:2, :] += jnp.sum(y * y, axis=0, keepdims=True)


def _edge(nbr, h2d, wcat):
    co = wcat.shape[1]
    return pl.pallas_call(
        _edge_body,
        grid=(R // NBE,),
        in_specs=[
            pl.BlockSpec((NBE, K, CP), lambda i: (i, 0, 0)),
            pl.BlockSpec((NBE, CP), lambda i: (i, 0)),
            pl.BlockSpec((2 * CP, co), lambda i: (0, 0)),
        ],
        out_specs=[
            pl.BlockSpec((NBE, co), lambda i: (i, 0)),
            pl.BlockSpec((8, co), lambda i: (0, 0)),
        ],
        out_shape=[
            jax.ShapeDtypeStruct((R, co), _f32),
            jax.ShapeDtypeStruct((8, co), _f32),
        ],
    )(nbr, h2d, wcat)


def _norm_body(m_ref, st_ref, g_ref, b_ref, o_ref):
    bnk = _f32(R * K)
    mean = st_ref[0:1, :] / bnk
    var = st_ref[1:2, :] / bnk - mean * mean
    y = g_ref[...] * (m_ref[...] - mean) / jnp.sqrt(var + 1e-5) + b_ref[...]
    o_ref[...] = jnp.where(y > 0, y, 0.2 * y)


def _norm(m2d, st, gamma, beta):
    co = m2d.shape[1]
    return pl.pallas_call(
        _norm_body,
        grid=(R // NB,),
        in_specs=[
            pl.BlockSpec((NB, co), lambda i: (i, 0)),
            pl.BlockSpec((8, co), lambda i: (0, 0)),
            pl.BlockSpec((1, co), lambda i: (0, 0)),
            pl.BlockSpec((1, co), lambda i: (0, 0)),
        ],
        out_specs=pl.BlockSpec((NB, co), lambda i: (i, 0)),
        out_shape=jax.ShapeDtypeStruct((R, co), _f32),
    )(m2d, st, gamma, beta)


def _final_body(h1_ref, h2_ref, h3_ref, h4_ref, w1_ref, w2_ref, w3_ref,
                w4_ref, bf_ref, o_ref):
    b = pl.program_id(0)
    i = pl.program_id(1)
    dn = (((1,), (0,)), ((), ()))
    y = lax.dot_general(h1_ref[...].astype(_bf16), w1_ref[...], dn,
                        preferred_element_type=_f32)
    y += lax.dot_general(h2_ref[...].astype(_bf16), w2_ref[...], dn,
                         preferred_element_type=_f32)
    y += lax.dot_general(h3_ref[...].astype(_bf16), w3_ref[...], dn,
                         preferred_element_type=_f32)
    y += lax.dot_general(h4_ref[...].astype(_bf16), w4_ref[...], dn,
                         preferred_element_type=_f32)
    y += bf_ref[...]
    part = jnp.max(y, axis=0, keepdims=True)

    @pl.when(i == 0)
    def _():
        o_ref[pl.ds(b, 1), :] = part

    @pl.when(i != 0)
    def _():
        o_ref[pl.ds(b, 1), :] = jnp.maximum(o_ref[pl.ds(b, 1), :], part)


def _final(hs, wfs, bf_row):
    nblk = N // NB
    in_specs = []
    args = []
    for h in hs:
        co = h.shape[1]
        in_specs.append(
            pl.BlockSpec((NB, co), lambda b, i: (b * nblk + i, 0)))
        args.append(h)
    for w in wfs:
        ci = w.shape[0]
        in_specs.append(pl.BlockSpec((ci, 1024), lambda b, i: (0, 0)))
        args.append(w)
    in_specs.append(pl.BlockSpec((1, 1024), lambda b, i: (0, 0)))
    args.append(bf_row)
    return pl.pallas_call(
        _final_body,
        grid=(B, nblk),
        in_specs=in_specs,
        out_specs=pl.BlockSpec((B, 1024), lambda b, i: (0, 0)),
        out_shape=jax.ShapeDtypeStruct((B, 1024), _f32),
    )(*args)


def kernel(x, W0, gamma0, beta0, W1, gamma1, beta1, W2, gamma2, beta2,
           W3, gamma3, beta3, Wf, bf):
    layers = [(W0, gamma0, beta0), (W1, gamma1, beta1), (W2, gamma2, beta2),
              (W3, gamma3, beta3)]
    h3d = jnp.pad(x, ((0, 0), (0, 0), (0, CP - 3)))
    ci_real = 3
    hs = []
    cos = []
    for W, gamma, beta in layers:
        co = W.shape[0]
        cp = max(co, CP)
        wa = jnp.pad(W[:, :ci_real].T, ((0, CP - ci_real), (0, cp - co)))
        wb = jnp.pad(W[:, ci_real:].T, ((0, CP - ci_real), (0, cp - co)))
        wcat = jnp.concatenate([wa, wb], axis=0).astype(_bf16)
        h2d = h3d.reshape(R, CP)
        dk = _prep(h3d)
        nbr = _sc_select_gather(dk.reshape(R, N), h2d)
        m2d, st = _edge(nbr, h2d, wcat)
        gp = jnp.pad(gamma, (0, cp - co)).reshape(1, cp)
        bp = jnp.pad(beta, (0, cp - co)).reshape(1, cp)
        hn = _norm(m2d, st, gp, bp)
        hs.append(hn)
        cos.append(co)
        if cp > CP:
            break
        h3d = hn.reshape(B, N, cp)
        ci_real = co
    ofs = 0
    wfs = []
    for hh, co in zip(hs, cos):
        cp = hh.shape[1]
        wfs.append(jnp.pad(Wf[:, ofs:ofs + co].T,
                           ((0, cp - co), (0, 0))).astype(_bf16))
        ofs += co
    return _final(hs, wfs, bf.reshape(1, 1024))


# speedup vs baseline: 6.8667x; 1.3784x over previous
import functools

import jax
import jax.numpy as jnp
from jax import lax
from jax.experimental import pallas as pl
from jax.experimental.pallas import tpu as pltpu
from jax.experimental.pallas import tpu_sc as plsc

K = 40
B = 4
N = 1024
R = B * N
CP = 128
NB = 256
NBE = 128
NTILES = 32
RPT = R // NTILES

_i32 = jnp.int32
_f32 = jnp.float32
_bf16 = jnp.bfloat16
INT_MIN = jnp.iinfo(jnp.int32).min


def _prep_body(hb_ref, hf_ref, dk_ref):
    hb = hb_ref[0]
    hf = hf_ref[0]
    dn = (((1,), (1,)), ((), ()))
    g = lax.dot_general(hb.astype(_bf16), hf.astype(_bf16), dn,
                        preferred_element_type=_f32)
    inner = -2.0 * g
    xxb = jnp.sum(hb * hb, axis=1, keepdims=True)
    sq = hf * hf
    ones = jnp.ones((8, CP), _f32)
    xx8 = lax.dot_general(ones, sq, dn, preferred_element_type=_f32,
                          precision=lax.Precision.HIGHEST)
    d = (-xxb - inner) - xx8[0:1, :]
    u = lax.bitcast_convert_type(d, _i32)
    key = jnp.where(u >= 0, u, u ^ jnp.int32(0x7FFFFFFF))
    dk_ref[...] = key[None]


def _prep(h):
    grid = (B, N // NB)
    return pl.pallas_call(
        _prep_body,
        grid=grid,
        in_specs=[
            pl.BlockSpec((1, NB, CP), lambda b, i: (b, i, 0)),
            pl.BlockSpec((1, N, CP), lambda b, i: (b, 0, 0)),
        ],
        out_specs=pl.BlockSpec((1, NB, N), lambda b, i: (b, i, 0)),
        out_shape=jax.ShapeDtypeStruct((B, N, N), _i32),
    )(h, h)


def _sc_body(dk_hbm, h_hbm, nbr_hbm, dbufA, dbufB, dbufTA, dbufTB, idxA,
             idxB, gbufA, gbufB, semDA, semDB, semGA, semGB, semWA, semWB):
    wid = lax.axis_index("s") * 2 + lax.axis_index("c")
    iota = lax.iota(_i32, 16)
    j0 = iota
    j1 = iota + 16
    j2 = iota + 32
    j3 = iota + 48
    base_r = wid * RPT

    def select_pair(ra, rb):
        bofsA = (ra // N) * N
        bofsB = (rb // N) * N

        def init_j(j, vv):
            va, vb = vv
            chA = dbufA[pl.ds(j * 16, 16)]
            chB = dbufB[pl.ds(j * 16, 16)]
            plsc.store_scatter(dbufTA, [iota * 64 + j], chA)
            plsc.store_scatter(dbufTB, [iota * 64 + j], chB)
            return (jnp.maximum(va, chA), jnp.maximum(vb, chB))

        neg = jnp.zeros((16,), _i32) + INT_MIN
        va, vb = lax.fori_loop(0, 64, init_j, (neg, neg), unroll=4)

        def one(dbufT, idxbuf, v, rowmax, bofs, t):
            lane = jnp.min(jnp.where(v == rowmax, iota, 16))
            base = lane * 64
            c0 = dbufT[pl.ds(base, 16)]
            c1 = dbufT[pl.ds(base + 16, 16)]
            c2 = dbufT[pl.ds(base + 32, 16)]
            c3 = dbufT[pl.ds(base + 48, 16)]
            cand = jnp.minimum(
                jnp.minimum(jnp.where(c0 == rowmax, j0, 64),
                            jnp.where(c1 == rowmax, j1, 64)),
                jnp.minimum(jnp.where(c2 == rowmax, j2, 64),
                            jnp.where(c3 == rowmax, j3, 64)))
            jstar = jnp.min(cand)
            gidx = jstar * 16 + lane
            zero = jnp.zeros((16,), _i32)
            plsc.store_scatter(idxbuf, [zero + t], zero + (gidx + bofs),
                               mask=iota == 0)
            plsc.store_scatter(dbufT, [zero + (base + jstar)], zero + INT_MIN,
                               mask=iota == 0)
            m0 = jnp.where(j0 == jstar, INT_MIN, c0)
            m1 = jnp.where(j1 == jstar, INT_MIN, c1)
            m2 = jnp.where(j2 == jstar, INT_MIN, c2)
            m3 = jnp.where(j3 == jstar, INT_MIN, c3)
            newmax = jnp.max(jnp.maximum(jnp.maximum(m0, m1),
                                         jnp.maximum(m2, m3)))
            return jnp.where(iota == lane, newmax, v)

        def sel_t(t, vv):
            va, vb = vv
            rmA = jnp.max(va)
            rmB = jnp.max(vb)
            va = one(dbufTA, idxA, va, rmA, bofsA, t)
            vb = one(dbufTB, idxB, vb, rmB, bofsB, t)
            return (va, vb)

        lax.fori_loop(0, K, sel_t, (va, vb))

    pltpu.async_copy(dk_hbm.at[base_r], dbufA, semDA)
    pltpu.async_copy(dk_hbm.at[base_r + 1], dbufB, semDB)
    NP = RPT // 2

    def pair_body(i, carry):
        ra = base_r + 2 * i
        rb = ra + 1

        @pl.when(i > 0)
        def _():
            pltpu.make_async_copy(h_hbm.at[idxA], gbufA, semGA).wait()
            pltpu.async_copy(gbufA, nbr_hbm.at[ra - 2], semWA)
            pltpu.make_async_copy(h_hbm.at[idxB], gbufB, semGB).wait()
            pltpu.async_copy(gbufB, nbr_hbm.at[rb - 2], semWB)

        pltpu.make_async_copy(dk_hbm.at[ra], dbufA, semDA).wait()
        pltpu.make_async_copy(dk_hbm.at[rb], dbufB, semDB).wait()
        select_pair(ra, rb)

        @pl.when(i > 0)
        def _():
            pltpu.make_async_copy(gbufA, nbr_hbm.at[ra - 2], semWA).wait()
            pltpu.make_async_copy(gbufB, nbr_hbm.at[rb - 2], semWB).wait()

        pltpu.async_copy(h_hbm.at[idxA], gbufA, semGA)
        pltpu.async_copy(h_hbm.at[idxB], gbufB, semGB)

        @pl.when(i < NP - 1)
        def _():
            pltpu.async_copy(dk_hbm.at[ra + 2], dbufA, semDA)
            pltpu.async_copy(dk_hbm.at[rb + 2], dbufB, semDB)

        return carry

    lax.fori_loop(0, NP, pair_body, 0)
    last = base_r + RPT - 2
    pltpu.make_async_copy(h_hbm.at[idxA], gbufA, semGA).wait()
    pltpu.async_copy(gbufA, nbr_hbm.at[last], semWA)
    pltpu.make_async_copy(h_hbm.at[idxB], gbufB, semGB).wait()
    pltpu.async_copy(gbufB, nbr_hbm.at[last + 1], semWB)
    pltpu.make_async_copy(gbufA, nbr_hbm.at[last], semWA).wait()
    pltpu.make_async_copy(gbufB, nbr_hbm.at[last + 1], semWB).wait()


def _sc_select_gather(dk2d, h2d):
    fn = pl.kernel(
        _sc_body,
        out_type=jax.ShapeDtypeStruct((R, K, CP), _f32),
        compiler_params=pltpu.CompilerParams(needs_layout_passes=False),
        mesh=plsc.VectorSubcoreMesh(core_axis_name="c", subcore_axis_name="s"),
        scratch_types=[
            pltpu.VMEM((N,), _i32),
            pltpu.VMEM((N,), _i32),
            pltpu.VMEM((N,), _i32),
            pltpu.VMEM((N,), _i32),
            pltpu.VMEM((K,), _i32),
            pltpu.VMEM((K,), _i32),
            pltpu.VMEM((K, CP), _f32),
            pltpu.VMEM((K, CP), _f32),
            pltpu.SemaphoreType.DMA,
            pltpu.SemaphoreType.DMA,
            pltpu.SemaphoreType.DMA,
            pltpu.SemaphoreType.DMA,
            pltpu.SemaphoreType.DMA,
            pltpu.SemaphoreType.DMA,
        ],
    )
    return fn(dk2d, h2d)


def _edge_body(nbr_ref, h_ref, w_ref, m_ref, st_ref):
    i = pl.program_id(0)
    nbr = nbr_ref[...]
    ctr = h_ref[...]
    ctr3 = jnp.broadcast_to(ctr[:, None, :], nbr.shape)
    a = (nbr - ctr3).astype(_bf16)
    bb = ctr3.astype(_bf16)
    feat = jnp.concatenate([a, bb], axis=-1).reshape(NBE * K, 2 * CP)
    dn = (((1,), (0,)), ((), ()))
    y = lax.dot_general(feat, w_ref[...], dn, preferred_element_type=_f32)
    co = y.shape[1]
    m_ref[...] = jnp.max(y.reshape(NBE, K, co), axis=1)

    @pl.when(i == 0)
    def _():
        st_ref[...] = jnp.zeros_like(st_ref)

    st_ref[0:1, :] += jnp.sum(y, axis=0, keepdims=True)
    st_ref[1:2, :] += jnp.sum(y * y, axis=0, keepdims=True)


def _edge(nbr, h2d, wcat):
    co = wcat.shape[1]
    return pl.pallas_call(
        _edge_body,
        grid=(R // NBE,),
        in_specs=[
            pl.BlockSpec((NBE, K, CP), lambda i: (i, 0, 0)),
            pl.BlockSpec((NBE, CP), lambda i: (i, 0)),
            pl.BlockSpec((2 * CP, co), lambda i: (0, 0)),
        ],
        out_specs=[
            pl.BlockSpec((NBE, co), lambda i: (i, 0)),
            pl.BlockSpec((8, co), lambda i: (0, 0)),
        ],
        out_shape=[
            jax.ShapeDtypeStruct((R, co), _f32),
            jax.ShapeDtypeStruct((8, co), _f32),
        ],
    )(nbr, h2d, wcat)


def _norm_body(m_ref, st_ref, g_ref, b_ref, o_ref):
    bnk = _f32(R * K)
    mean = st_ref[0:1, :] / bnk
    var = st_ref[1:2, :] / bnk - mean * mean
    y = g_ref[...] * (m_ref[...] - mean) / jnp.sqrt(var + 1e-5) + b_ref[...]
    o_ref[...] = jnp.where(y > 0, y, 0.2 * y)


def _norm(m2d, st, gamma, beta):
    co = m2d.shape[1]
    return pl.pallas_call(
        _norm_body,
        grid=(R // NB,),
        in_specs=[
            pl.BlockSpec((NB, co), lambda i: (i, 0)),
            pl.BlockSpec((8, co), lambda i: (0, 0)),
            pl.BlockSpec((1, co), lambda i: (0, 0)),
            pl.BlockSpec((1, co), lambda i: (0, 0)),
        ],
        out_specs=pl.BlockSpec((NB, co), lambda i: (i, 0)),
        out_shape=jax.ShapeDtypeStruct((R, co), _f32),
    )(m2d, st, gamma, beta)


def _final_body(h1_ref, h2_ref, h3_ref, h4_ref, w1_ref, w2_ref, w3_ref,
                w4_ref, bf_ref, o_ref):
    b = pl.program_id(0)
    i = pl.program_id(1)
    dn = (((1,), (0,)), ((), ()))
    y = lax.dot_general(h1_ref[...].astype(_bf16), w1_ref[...], dn,
                        preferred_element_type=_f32)
    y += lax.dot_general(h2_ref[...].astype(_bf16), w2_ref[...], dn,
                         preferred_element_type=_f32)
    y += lax.dot_general(h3_ref[...].astype(_bf16), w3_ref[...], dn,
                         preferred_element_type=_f32)
    y += lax.dot_general(h4_ref[...].astype(_bf16), w4_ref[...], dn,
                         preferred_element_type=_f32)
    y += bf_ref[...]
    part = jnp.max(y, axis=0, keepdims=True)

    @pl.when(i == 0)
    def _():
        o_ref[pl.ds(b, 1), :] = part

    @pl.when(i != 0)
    def _():
        o_ref[pl.ds(b, 1), :] = jnp.maximum(o_ref[pl.ds(b, 1), :], part)


def _final(hs, wfs, bf_row):
    nblk = N // NB
    in_specs = []
    args = []
    for h in hs:
        co = h.shape[1]
        in_specs.append(
            pl.BlockSpec((NB, co), lambda b, i: (b * nblk + i, 0)))
        args.append(h)
    for w in wfs:
        ci = w.shape[0]
        in_specs.append(pl.BlockSpec((ci, 1024), lambda b, i: (0, 0)))
        args.append(w)
    in_specs.append(pl.BlockSpec((1, 1024), lambda b, i: (0, 0)))
    args.append(bf_row)
    return pl.pallas_call(
        _final_body,
        grid=(B, nblk),
        in_specs=in_specs,
        out_specs=pl.BlockSpec((B, 1024), lambda b, i: (0, 0)),
        out_shape=jax.ShapeDtypeStruct((B, 1024), _f32),
    )(*args)


def kernel(x, W0, gamma0, beta0, W1, gamma1, beta1, W2, gamma2, beta2,
           W3, gamma3, beta3, Wf, bf):
    layers = [(W0, gamma0, beta0), (W1, gamma1, beta1), (W2, gamma2, beta2),
              (W3, gamma3, beta3)]
    h3d = jnp.pad(x, ((0, 0), (0, 0), (0, CP - 3)))
    ci_real = 3
    hs = []
    cos = []
    for W, gamma, beta in layers:
        co = W.shape[0]
        cp = max(co, CP)
        wa = jnp.pad(W[:, :ci_real].T, ((0, CP - ci_real), (0, cp - co)))
        wb = jnp.pad(W[:, ci_real:].T, ((0, CP - ci_real), (0, cp - co)))
        wcat = jnp.concatenate([wa, wb], axis=0).astype(_bf16)
        h2d = h3d.reshape(R, CP)
        dk = _prep(h3d)
        nbr = _sc_select_gather(dk.reshape(R, N), h2d)
        m2d, st = _edge(nbr, h2d, wcat)
        gp = jnp.pad(gamma, (0, cp - co)).reshape(1, cp)
        bp = jnp.pad(beta, (0, cp - co)).reshape(1, cp)
        hn = _norm(m2d, st, gp, bp)
        hs.append(hn)
        cos.append(co)
        if cp > CP:
            break
        h3d = hn.reshape(B, N, cp)
        ci_real = co
    ofs = 0
    wfs = []
    for hh, co in zip(hs, cos):
        cp = hh.shape[1]
        wfs.append(jnp.pad(Wf[:, ofs:ofs + co].T,
                           ((0, cp - co), (0, 0))).astype(_bf16))
        ofs += co
    return _final(hs, wfs, bf.reshape(1, 1024))


# speedup vs baseline: 8.1256x; 1.1833x over previous
import functools

import jax
import jax.numpy as jnp
from jax import lax
from jax.experimental import pallas as pl
from jax.experimental.pallas import tpu as pltpu
from jax.experimental.pallas import tpu_sc as plsc

K = 40
B = 4
N = 1024
R = B * N
CP = 128
NB = 256
NBE = 128
NTILES = 32
RPT = R // NTILES

_i32 = jnp.int32
_f32 = jnp.float32
_bf16 = jnp.bfloat16
INT_MIN = jnp.iinfo(jnp.int32).min


def _prep_body(hb_ref, hf_ref, dk_ref):
    hb = hb_ref[0]
    hf = hf_ref[0]
    dn = (((1,), (1,)), ((), ()))
    g = lax.dot_general(hb.astype(_bf16), hf.astype(_bf16), dn,
                        preferred_element_type=_f32)
    inner = -2.0 * g
    xxb = jnp.sum(hb * hb, axis=1, keepdims=True)
    sq = hf * hf
    ones = jnp.ones((8, CP), _f32)
    xx8 = lax.dot_general(ones, sq, dn, preferred_element_type=_f32,
                          precision=lax.Precision.HIGHEST)
    d = (-xxb - inner) - xx8[0:1, :]
    u = lax.bitcast_convert_type(d, _i32)
    key = jnp.where(u >= 0, u, u ^ jnp.int32(0x7FFFFFFF))
    dk_ref[...] = key[None]


def _prep(h):
    grid = (B, N // NB)
    return pl.pallas_call(
        _prep_body,
        grid=grid,
        in_specs=[
            pl.BlockSpec((1, NB, CP), lambda b, i: (b, i, 0)),
            pl.BlockSpec((1, N, CP), lambda b, i: (b, 0, 0)),
        ],
        out_specs=pl.BlockSpec((1, NB, N), lambda b, i: (b, i, 0)),
        out_shape=jax.ShapeDtypeStruct((B, N, N), _i32),
    )(h, h)


QW = 4


def _sc_body(dk_hbm, h_hbm, nbr_hbm, *scr):
    dbuf = scr[0:QW]
    dbufT = scr[QW:2 * QW]
    idx = scr[2 * QW:3 * QW]
    gbuf = scr[3 * QW:4 * QW]
    semD = scr[4 * QW:5 * QW]
    semG = scr[5 * QW:6 * QW]
    semW = scr[6 * QW:7 * QW]
    wid = lax.axis_index("s") * 2 + lax.axis_index("c")
    iota = lax.iota(_i32, 16)
    j0 = iota
    j1 = iota + 16
    j2 = iota + 32
    j3 = iota + 48
    base_r = wid * RPT

    def select_group(r0):
        bofs = [((r0 + q) // N) * N for q in range(QW)]

        def init_j(j, vv):
            out = []
            for q in range(QW):
                ch = dbuf[q][pl.ds(j * 16, 16)]
                plsc.store_scatter(dbufT[q], [iota * 64 + j], ch)
                out.append(jnp.maximum(vv[q], ch))
            return tuple(out)

        neg = jnp.zeros((16,), _i32) + INT_MIN
        vs = lax.fori_loop(0, 64, init_j, (neg,) * QW, unroll=4)

        def one(q, v, rowmax, t):
            lane = jnp.min(jnp.where(v == rowmax, iota, 16))
            base = lane * 64
            c0 = dbufT[q][pl.ds(base, 16)]
            c1 = dbufT[q][pl.ds(base + 16, 16)]
            c2 = dbufT[q][pl.ds(base + 32, 16)]
            c3 = dbufT[q][pl.ds(base + 48, 16)]
            cand = jnp.minimum(
                jnp.minimum(jnp.where(c0 == rowmax, j0, 64),
                            jnp.where(c1 == rowmax, j1, 64)),
                jnp.minimum(jnp.where(c2 == rowmax, j2, 64),
                            jnp.where(c3 == rowmax, j3, 64)))
            jstar = jnp.min(cand)
            gidx = jstar * 16 + lane
            zero = jnp.zeros((16,), _i32)
            plsc.store_scatter(idx[q], [zero + t], zero + (gidx + bofs[q]),
                               mask=iota == 0)
            plsc.store_scatter(dbufT[q], [zero + (base + jstar)],
                               zero + INT_MIN, mask=iota == 0)
            m0 = jnp.where(j0 == jstar, INT_MIN, c0)
            m1 = jnp.where(j1 == jstar, INT_MIN, c1)
            m2 = jnp.where(j2 == jstar, INT_MIN, c2)
            m3 = jnp.where(j3 == jstar, INT_MIN, c3)
            newmax = jnp.max(jnp.maximum(jnp.maximum(m0, m1),
                                         jnp.maximum(m2, m3)))
            return jnp.where(iota == lane, newmax, v)

        def sel_t(t, vv):
            rms = [jnp.max(vv[q]) for q in range(QW)]
            return tuple(one(q, vv[q], rms[q], t) for q in range(QW))

        lax.fori_loop(0, K, sel_t, vs)

    for q in range(QW):
        pltpu.async_copy(dk_hbm.at[base_r + q], dbuf[q], semD[q])
    NG = RPT // QW

    def group_body(i, carry):
        r0 = base_r + QW * i

        @pl.when(i > 0)
        def _():
            for q in range(QW):
                pltpu.make_async_copy(h_hbm.at[idx[q]], gbuf[q],
                                      semG[q]).wait()
                pltpu.async_copy(gbuf[q], nbr_hbm.at[r0 - QW + q], semW[q])

        for q in range(QW):
            pltpu.make_async_copy(dk_hbm.at[r0 + q], dbuf[q], semD[q]).wait()
        select_group(r0)

        @pl.when(i > 0)
        def _():
            for q in range(QW):
                pltpu.make_async_copy(gbuf[q], nbr_hbm.at[r0 - QW + q],
                                      semW[q]).wait()

        for q in range(QW):
            pltpu.async_copy(h_hbm.at[idx[q]], gbuf[q], semG[q])

        @pl.when(i < NG - 1)
        def _():
            for q in range(QW):
                pltpu.async_copy(dk_hbm.at[r0 + QW + q], dbuf[q], semD[q])

        return carry

    lax.fori_loop(0, NG, group_body, 0)
    last = base_r + RPT - QW
    for q in range(QW):
        pltpu.make_async_copy(h_hbm.at[idx[q]], gbuf[q], semG[q]).wait()
        pltpu.async_copy(gbuf[q], nbr_hbm.at[last + q], semW[q])
    for q in range(QW):
        pltpu.make_async_copy(gbuf[q], nbr_hbm.at[last + q], semW[q]).wait()


def _sc_select_gather(dk2d, h2d):
    fn = pl.kernel(
        _sc_body,
        out_type=jax.ShapeDtypeStruct((R, K, CP), _f32),
        compiler_params=pltpu.CompilerParams(needs_layout_passes=False),
        mesh=plsc.VectorSubcoreMesh(core_axis_name="c", subcore_axis_name="s"),
        scratch_types=(
            [pltpu.VMEM((N,), _i32)] * QW
            + [pltpu.VMEM((N,), _i32)] * QW
            + [pltpu.VMEM((K,), _i32)] * QW
            + [pltpu.VMEM((K, CP), _f32)] * QW
            + [pltpu.SemaphoreType.DMA] * (3 * QW)
        ),
    )
    return fn(dk2d, h2d)


def _edge_body(nbr_ref, h_ref, w_ref, m_ref, st_ref):
    i = pl.program_id(0)
    nbr = nbr_ref[...]
    ctr = h_ref[...]
    ctr3 = jnp.broadcast_to(ctr[:, None, :], nbr.shape)
    a = (nbr - ctr3).astype(_bf16)
    bb = ctr3.astype(_bf16)
    feat = jnp.concatenate([a, bb], axis=-1).reshape(NBE * K, 2 * CP)
    dn = (((1,), (0,)), ((), ()))
    y = lax.dot_general(feat, w_ref[...], dn, preferred_element_type=_f32)
    co = y.shape[1]
    m_ref[...] = jnp.max(y.reshape(NBE, K, co), axis=1)

    @pl.when(i == 0)
    def _():
        st_ref[...] = jnp.zeros_like(st_ref)

    st_ref[0:1, :] += jnp.sum(y, axis=0, keepdims=True)
    st_ref[1:2, :] += jnp.sum(y * y, axis=0, keepdims=True)


def _edge(nbr, h2d, wcat):
    co = wcat.shape[1]
    return pl.pallas_call(
        _edge_body,
        grid=(R // NBE,),
        in_specs=[
            pl.BlockSpec((NBE, K, CP), lambda i: (i, 0, 0)),
            pl.BlockSpec((NBE, CP), lambda i: (i, 0)),
            pl.BlockSpec((2 * CP, co), lambda i: (0, 0)),
        ],
        out_specs=[
            pl.BlockSpec((NBE, co), lambda i: (i, 0)),
            pl.BlockSpec((8, co), lambda i: (0, 0)),
        ],
        out_shape=[
            jax.ShapeDtypeStruct((R, co), _f32),
            jax.ShapeDtypeStruct((8, co), _f32),
        ],
    )(nbr, h2d, wcat)


def _norm_body(m_ref, st_ref, g_ref, b_ref, o_ref):
    bnk = _f32(R * K)
    mean = st_ref[0:1, :] / bnk
    var = st_ref[1:2, :] / bnk - mean * mean
    y = g_ref[...] * (m_ref[...] - mean) / jnp.sqrt(var + 1e-5) + b_ref[...]
    o_ref[...] = jnp.where(y > 0, y, 0.2 * y)


def _norm(m2d, st, gamma, beta):
    co = m2d.shape[1]
    return pl.pallas_call(
        _norm_body,
        grid=(R // NB,),
        in_specs=[
            pl.BlockSpec((NB, co), lambda i: (i, 0)),
            pl.BlockSpec((8, co), lambda i: (0, 0)),
            pl.BlockSpec((1, co), lambda i: (0, 0)),
            pl.BlockSpec((1, co), lambda i: (0, 0)),
        ],
        out_specs=pl.BlockSpec((NB, co), lambda i: (i, 0)),
        out_shape=jax.ShapeDtypeStruct((R, co), _f32),
    )(m2d, st, gamma, beta)


def _final_body(h1_ref, h2_ref, h3_ref, h4_ref, w1_ref, w2_ref, w3_ref,
                w4_ref, bf_ref, o_ref):
    b = pl.program_id(0)
    i = pl.program_id(1)
    dn = (((1,), (0,)), ((), ()))
    y = lax.dot_general(h1_ref[...].astype(_bf16), w1_ref[...], dn,
                        preferred_element_type=_f32)
    y += lax.dot_general(h2_ref[...].astype(_bf16), w2_ref[...], dn,
                         preferred_element_type=_f32)
    y += lax.dot_general(h3_ref[...].astype(_bf16), w3_ref[...], dn,
                         preferred_element_type=_f32)
    y += lax.dot_general(h4_ref[...].astype(_bf16), w4_ref[...], dn,
                         preferred_element_type=_f32)
    y += bf_ref[...]
    part = jnp.max(y, axis=0, keepdims=True)

    @pl.when(i == 0)
    def _():
        o_ref[pl.ds(b, 1), :] = part

    @pl.when(i != 0)
    def _():
        o_ref[pl.ds(b, 1), :] = jnp.maximum(o_ref[pl.ds(b, 1), :], part)


def _final(hs, wfs, bf_row):
    nblk = N // NB
    in_specs = []
    args = []
    for h in hs:
        co = h.shape[1]
        in_specs.append(
            pl.BlockSpec((NB, co), lambda b, i: (b * nblk + i, 0)))
        args.append(h)
    for w in wfs:
        ci = w.shape[0]
        in_specs.append(pl.BlockSpec((ci, 1024), lambda b, i: (0, 0)))
        args.append(w)
    in_specs.append(pl.BlockSpec((1, 1024), lambda b, i: (0, 0)))
    args.append(bf_row)
    return pl.pallas_call(
        _final_body,
        grid=(B, nblk),
        in_specs=in_specs,
        out_specs=pl.BlockSpec((B, 1024), lambda b, i: (0, 0)),
        out_shape=jax.ShapeDtypeStruct((B, 1024), _f32),
    )(*args)


def kernel(x, W0, gamma0, beta0, W1, gamma1, beta1, W2, gamma2, beta2,
           W3, gamma3, beta3, Wf, bf):
    layers = [(W0, gamma0, beta0), (W1, gamma1, beta1), (W2, gamma2, beta2),
              (W3, gamma3, beta3)]
    h3d = jnp.pad(x, ((0, 0), (0, 0), (0, CP - 3)))
    ci_real = 3
    hs = []
    cos = []
    for W, gamma, beta in layers:
        co = W.shape[0]
        cp = max(co, CP)
        wa = jnp.pad(W[:, :ci_real].T, ((0, CP - ci_real), (0, cp - co)))
        wb = jnp.pad(W[:, ci_real:].T, ((0, CP - ci_real), (0, cp - co)))
        wcat = jnp.concatenate([wa, wb], axis=0).astype(_bf16)
        h2d = h3d.reshape(R, CP)
        dk = _prep(h3d)
        nbr = _sc_select_gather(dk.reshape(R, N), h2d)
        m2d, st = _edge(nbr, h2d, wcat)
        gp = jnp.pad(gamma, (0, cp - co)).reshape(1, cp)
        bp = jnp.pad(beta, (0, cp - co)).reshape(1, cp)
        hn = _norm(m2d, st, gp, bp)
        hs.append(hn)
        cos.append(co)
        if cp > CP:
            break
        h3d = hn.reshape(B, N, cp)
        ci_real = co
    ofs = 0
    wfs = []
    for hh, co in zip(hs, cos):
        cp = hh.shape[1]
        wfs.append(jnp.pad(Wf[:, ofs:ofs + co].T,
                           ((0, cp - co), (0, 0))).astype(_bf16))
        ofs += co
    return _final(hs, wfs, bf.reshape(1, 1024))
